# baseline (speedup 1.0000x reference)
import jax
import jax.numpy as jnp
from jax import lax
from jax.experimental import pallas as pl
from jax.experimental.pallas import tpu as pltpu

NQ = 4
K = 1024
D = 256
BN = 1024


def _prep_block(cb_ref, hi_ref, cat_ref, cn_ref):
    ones_row = jnp.ones((8, D), jnp.float32)
    for s in range(NQ):
        cb = cb_ref[s]
        hi = cb.astype(jnp.bfloat16)
        rem = cb - hi.astype(jnp.float32)
        mid = rem.astype(jnp.bfloat16)
        lo = (rem - mid.astype(jnp.float32)).astype(jnp.bfloat16)
        hi_ref[s] = hi
        cat_ref[s, :, 0:D] = hi
        cat_ref[s, :, D:2 * D] = mid
        cat_ref[s, :, 2 * D:] = lo
        cn_ref[s] = lax.dot_general(ones_row, cb * cb,
                                    (((1,), (1,)), ((), ())),
                                    preferred_element_type=jnp.float32,
                                    precision=lax.Precision.HIGHEST)


def _rvq_block(x_ref, hi_ref, cat_ref, cn_ref,
               q_ref, idx_ref, loss_ref):
    r0 = x_ref[...]
    r = r0
    rn = jnp.sum(r * r, axis=1, keepdims=True)
    qsum = jnp.zeros((BN, D), jnp.float32)
    iota_f = lax.broadcasted_iota(jnp.int32, (BN, K), 1).astype(jnp.float32)
    idx_cols = lax.broadcasted_iota(jnp.int32, (BN, 8), 1)
    idx_acc = jnp.zeros((BN, 8), jnp.int32)
    loss_rows = lax.broadcasted_iota(jnp.int32, (8, 128), 0)
    loss_cols = lax.broadcasted_iota(jnp.int32, (8, 128), 1)
    loss_contrib = jnp.zeros((8, 128), jnp.float32)

    for s in range(NQ):
        cb_hi = hi_ref[s]
        r2b = (2.0 * r).astype(jnp.bfloat16)
        m2 = lax.dot_general(r2b, cb_hi, (((1,), (1,)), ((), ())),
                             preferred_element_type=jnp.float32)
        d2 = (rn - m2) + cn_ref[s, 0:1, :]
        dmin = jnp.min(d2, axis=1, keepdims=True)
        idx_f = jnp.min(jnp.where(d2 == dmin, iota_f, jnp.float32(K)),
                        axis=1, keepdims=True)
        onehot = (iota_f == idx_f).astype(jnp.bfloat16)
        dn = (((1,), (0,)), ((), ()))
        u = lax.dot_general(onehot, cat_ref[s], dn,
                            preferred_element_type=jnp.float32)
        q = (u[:, 0:D] + u[:, D:2 * D]) + u[:, 2 * D:]
        r = r - q
        qsum = qsum + q
        rn = jnp.sum(r * r, axis=1, keepdims=True)
        loss_s = jnp.sum(rn)
        loss_contrib = loss_contrib + loss_s * jnp.where(
            (loss_rows == 0) & (loss_cols == s), 1.0, 0.0)
        idx_acc = idx_acc + jnp.where(idx_cols == s,
                                      idx_f.astype(jnp.int32), 0)

    q_ref[...] = r0 + (qsum - r0)
    idx_ref[...] = idx_acc
    loss_ref[...] = loss_contrib[None]


def kernel(x, codebooks):
    B, T, Dd = x.shape
    N = B * T
    G = N // BN
    flat = x.reshape(N, Dd)

    cb_hi, cb_cat, cn = pl.pallas_call(
        _prep_block,
        in_specs=[pl.BlockSpec((NQ, K, D), lambda: (0, 0, 0))],
        out_specs=[
            pl.BlockSpec((NQ, K, D), lambda: (0, 0, 0)),
            pl.BlockSpec((NQ, K, 3 * D), lambda: (0, 0, 0)),
            pl.BlockSpec((NQ, 8, K), lambda: (0, 0, 0)),
        ],
        out_shape=[
            jax.ShapeDtypeStruct((NQ, K, D), jnp.bfloat16),
            jax.ShapeDtypeStruct((NQ, K, 3 * D), jnp.bfloat16),
            jax.ShapeDtypeStruct((NQ, 8, K), jnp.float32),
        ],
    )(codebooks)

    q_flat, idx_pack, loss_sums = pl.pallas_call(
        _rvq_block,
        grid=(G,),
        in_specs=[
            pl.BlockSpec((BN, D), lambda i: (i, 0)),
            pl.BlockSpec((NQ, K, D), lambda i: (0, 0, 0)),
            pl.BlockSpec((NQ, K, 3 * D), lambda i: (0, 0, 0)),
            pl.BlockSpec((NQ, 8, K), lambda i: (0, 0, 0)),
        ],
        out_specs=[
            pl.BlockSpec((BN, D), lambda i: (i, 0)),
            pl.BlockSpec((BN, 8), lambda i: (i, 0)),
            pl.BlockSpec((1, 8, 128), lambda i: (i, 0, 0)),
        ],
        out_shape=[
            jax.ShapeDtypeStruct((N, D), jnp.float32),
            jax.ShapeDtypeStruct((N, 8), jnp.int32),
            jax.ShapeDtypeStruct((G, 8, 128), jnp.float32),
        ],
        compiler_params=pltpu.CompilerParams(
            dimension_semantics=("arbitrary",),
        ),
    )(flat, cb_hi, cb_cat, cn)

    quantized = q_flat.reshape(B, T, Dd)
    indices = idx_pack[:, :NQ].reshape(B, T, NQ)
    vq_loss = 1.25 * jnp.sum(loss_sums[:, 0, :NQ]) / jnp.float32(N * Dd)
    losses = jnp.full((NQ,), vq_loss, dtype=jnp.float32)
    return quantized, indices, losses


# speedup vs baseline: 1.0319x; 1.0319x over previous
import jax
import jax.numpy as jnp
from jax import lax
from jax.experimental import pallas as pl
from jax.experimental.pallas import tpu as pltpu

NQ = 4
K = 1024
D = 256
BN = 1024


def _rvq_block(x_ref, cb_ref, q_ref, idx_ref, loss_ref,
               hi_ref, cat_ref, cn_ref):
    @pl.when(pl.program_id(0) == 0)
    def _prep():
        ones_row = jnp.ones((8, D), jnp.float32)
        for s in range(NQ):
            cb = cb_ref[s]
            hi = cb.astype(jnp.bfloat16)
            rem = cb - hi.astype(jnp.float32)
            mid = rem.astype(jnp.bfloat16)
            lo = (rem - mid.astype(jnp.float32)).astype(jnp.bfloat16)
            hi_ref[s] = hi
            cat_ref[s, :, 0:D] = hi
            cat_ref[s, :, D:2 * D] = mid
            cat_ref[s, :, 2 * D:] = lo
            cn_ref[s] = lax.dot_general(ones_row, cb * cb,
                                        (((1,), (1,)), ((), ())),
                                        preferred_element_type=jnp.float32,
                                        precision=lax.Precision.HIGHEST)

    r0 = x_ref[...]
    r = r0
    rn = jnp.sum(r * r, axis=1, keepdims=True)
    qsum = jnp.zeros((BN, D), jnp.float32)
    iota_f = lax.broadcasted_iota(jnp.int32, (BN, K), 1).astype(jnp.float32)
    idx_cols = lax.broadcasted_iota(jnp.int32, (BN, 8), 1)
    idx_acc = jnp.zeros((BN, 8), jnp.int32)
    loss_rows = lax.broadcasted_iota(jnp.int32, (8, 128), 0)
    loss_cols = lax.broadcasted_iota(jnp.int32, (8, 128), 1)
    loss_contrib = jnp.zeros((8, 128), jnp.float32)

    for s in range(NQ):
        cb_hi = hi_ref[s]
        r2b = (2.0 * r).astype(jnp.bfloat16)
        m2 = lax.dot_general(r2b, cb_hi, (((1,), (1,)), ((), ())),
                             preferred_element_type=jnp.float32)
        d2 = (rn - m2) + cn_ref[s, 0:1, :]
        dmin = jnp.min(d2, axis=1, keepdims=True)
        idx_f = jnp.min(jnp.where(d2 == dmin, iota_f, jnp.float32(K)),
                        axis=1, keepdims=True)
        onehot = (iota_f == idx_f).astype(jnp.bfloat16)
        dn = (((1,), (0,)), ((), ()))
        u = lax.dot_general(onehot, cat_ref[s], dn,
                            preferred_element_type=jnp.float32)
        q = (u[:, 0:D] + u[:, D:2 * D]) + u[:, 2 * D:]
        r = r - q
        qsum = qsum + q
        rn = jnp.sum(r * r, axis=1, keepdims=True)
        loss_s = jnp.sum(rn)
        loss_contrib = loss_contrib + loss_s * jnp.where(
            (loss_rows == 0) & (loss_cols == s), 1.0, 0.0)
        idx_acc = idx_acc + jnp.where(idx_cols == s,
                                      idx_f.astype(jnp.int32), 0)

    q_ref[...] = r0 + (qsum - r0)
    idx_ref[...] = idx_acc
    loss_ref[...] = loss_contrib[None]


def kernel(x, codebooks):
    B, T, Dd = x.shape
    N = B * T
    G = N // BN
    flat = x.reshape(N, Dd)

    q_flat, idx_pack, loss_sums = pl.pallas_call(
        _rvq_block,
        grid=(G,),
        in_specs=[
            pl.BlockSpec((BN, D), lambda i: (i, 0)),
            pl.BlockSpec((NQ, K, D), lambda i: (0, 0, 0)),
        ],
        out_specs=[
            pl.BlockSpec((BN, D), lambda i: (i, 0)),
            pl.BlockSpec((BN, 8), lambda i: (i, 0)),
            pl.BlockSpec((1, 8, 128), lambda i: (i, 0, 0)),
        ],
        out_shape=[
            jax.ShapeDtypeStruct((N, D), jnp.float32),
            jax.ShapeDtypeStruct((N, 8), jnp.int32),
            jax.ShapeDtypeStruct((G, 8, 128), jnp.float32),
        ],
        scratch_shapes=[
            pltpu.VMEM((NQ, K, D), jnp.bfloat16),
            pltpu.VMEM((NQ, K, 3 * D), jnp.bfloat16),
            pltpu.VMEM((NQ, 8, K), jnp.float32),
        ],
        compiler_params=pltpu.CompilerParams(
            dimension_semantics=("arbitrary",),
        ),
    )(flat, codebooks)

    quantized = q_flat.reshape(B, T, Dd)
    indices = idx_pack[:, :NQ].reshape(B, T, NQ)
    vq_loss = 1.25 * jnp.sum(loss_sums[:, 0, :NQ]) / jnp.float32(N * Dd)
    losses = jnp.full((NQ,), vq_loss, dtype=jnp.float32)
    return quantized, indices, losses


# speedup vs baseline: 1.0626x; 1.0297x over previous
import jax
import jax.numpy as jnp
from jax import lax
from jax.experimental import pallas as pl
from jax.experimental.pallas import tpu as pltpu

NQ = 4
K = 1024
D = 256
BN = 1024
HN = BN // 2


def _rvq_block(x_ref, cb_ref, q_ref, idx_ref, loss_ref,
               hi_ref, cat_ref, cn_ref):
    @pl.when(pl.program_id(0) == 0)
    def _prep():
        ones_row = jnp.ones((8, D), jnp.float32)
        for s in range(NQ):
            cb = cb_ref[s]
            hi = cb.astype(jnp.bfloat16)
            rem = cb - hi.astype(jnp.float32)
            mid = rem.astype(jnp.bfloat16)
            lo = (rem - mid.astype(jnp.float32)).astype(jnp.bfloat16)
            hi_ref[s] = hi
            cat_ref[s, :, 0:D] = hi
            cat_ref[s, :, D:2 * D] = mid
            cat_ref[s, :, 2 * D:] = lo
            cn_ref[s] = lax.dot_general(ones_row, cb * cb,
                                        (((1,), (1,)), ((), ())),
                                        preferred_element_type=jnp.float32,
                                        precision=lax.Precision.HIGHEST)

    iota_f = lax.broadcasted_iota(jnp.int32, (HN, K), 1).astype(jnp.float32)
    idx_cols = lax.broadcasted_iota(jnp.int32, (HN, 8), 1)

    def _chain(r0):
        r = r0
        rn = jnp.sum(r * r, axis=1, keepdims=True)
        qsum = jnp.zeros((HN, D), jnp.float32)
        idx_acc = jnp.zeros((HN, 8), jnp.int32)
        losses = []
        for s in range(NQ):
            r2b = (2.0 * r).astype(jnp.bfloat16)
            m2 = lax.dot_general(r2b, hi_ref[s], (((1,), (1,)), ((), ())),
                                 preferred_element_type=jnp.float32)
            d2 = (rn - m2) + cn_ref[s, 0:1, :]
            dmin = jnp.min(d2, axis=1, keepdims=True)
            idx_f = jnp.min(jnp.where(d2 == dmin, iota_f, jnp.float32(K)),
                            axis=1, keepdims=True)
            onehot = (iota_f == idx_f).astype(jnp.bfloat16)
            u = lax.dot_general(onehot, cat_ref[s], (((1,), (0,)), ((), ())),
                                preferred_element_type=jnp.float32)
            q = (u[:, 0:D] + u[:, D:2 * D]) + u[:, 2 * D:]
            r = r - q
            qsum = qsum + q
            rn = jnp.sum(r * r, axis=1, keepdims=True)
            losses.append(jnp.sum(rn))
            idx_acc = idx_acc + jnp.where(idx_cols == s,
                                          idx_f.astype(jnp.int32), 0)
        return qsum, idx_acc, losses

    r0a = x_ref[0:HN, :]
    r0b = x_ref[HN:BN, :]
    qsum_a, idx_a, loss_a = _chain(r0a)
    qsum_b, idx_b, loss_b = _chain(r0b)

    q_ref[0:HN, :] = r0a + (qsum_a - r0a)
    q_ref[HN:BN, :] = r0b + (qsum_b - r0b)
    idx_ref[0:HN, :] = idx_a
    idx_ref[HN:BN, :] = idx_b

    loss_rows = lax.broadcasted_iota(jnp.int32, (8, 128), 0)
    loss_cols = lax.broadcasted_iota(jnp.int32, (8, 128), 1)
    loss_contrib = jnp.zeros((8, 128), jnp.float32)
    for s in range(NQ):
        loss_contrib = loss_contrib + (loss_a[s] + loss_b[s]) * jnp.where(
            (loss_rows == 0) & (loss_cols == s), 1.0, 0.0)
    loss_ref[...] = loss_contrib[None]


def kernel(x, codebooks):
    B, T, Dd = x.shape
    N = B * T
    G = N // BN
    flat = x.reshape(N, Dd)

    q_flat, idx_pack, loss_sums = pl.pallas_call(
        _rvq_block,
        grid=(G,),
        in_specs=[
            pl.BlockSpec((BN, D), lambda i: (i, 0)),
            pl.BlockSpec((NQ, K, D), lambda i: (0, 0, 0)),
        ],
        out_specs=[
            pl.BlockSpec((BN, D), lambda i: (i, 0)),
            pl.BlockSpec((BN, 8), lambda i: (i, 0)),
            pl.BlockSpec((1, 8, 128), lambda i: (i, 0, 0)),
        ],
        out_shape=[
            jax.ShapeDtypeStruct((N, D), jnp.float32),
            jax.ShapeDtypeStruct((N, 8), jnp.int32),
            jax.ShapeDtypeStruct((G, 8, 128), jnp.float32),
        ],
        scratch_shapes=[
            pltpu.VMEM((NQ, K, D), jnp.bfloat16),
            pltpu.VMEM((NQ, K, 3 * D), jnp.bfloat16),
            pltpu.VMEM((NQ, 8, K), jnp.float32),
        ],
        compiler_params=pltpu.CompilerParams(
            dimension_semantics=("arbitrary",),
        ),
    )(flat, codebooks)

    quantized = q_flat.reshape(B, T, Dd)
    indices = idx_pack[:, :NQ].reshape(B, T, NQ)
    vq_loss = 1.25 * jnp.sum(loss_sums[:, 0, :NQ]) / jnp.float32(N * Dd)
    losses = jnp.full((NQ,), vq_loss, dtype=jnp.float32)
    return quantized, indices, losses


# speedup vs baseline: 1.1137x; 1.0481x over previous
import jax
import jax.numpy as jnp
from jax import lax
from jax.experimental import pallas as pl
from jax.experimental.pallas import tpu as pltpu

NQ = 4
K = 1024
D = 256
BN = 1536
HN = BN // 2


def _rvq_block(x_ref, cb_ref, q_ref, idx_ref, loss_ref,
               hi_ref, cat_ref, cn_ref):
    @pl.when(pl.program_id(0) == 0)
    def _prep():
        ones_row = jnp.ones((8, D), jnp.float32)
        for s in range(NQ):
            cb = cb_ref[s]
            hi = cb.astype(jnp.bfloat16)
            rem = cb - hi.astype(jnp.float32)
            mid = rem.astype(jnp.bfloat16)
            lo = (rem - mid.astype(jnp.float32)).astype(jnp.bfloat16)
            hi_ref[s] = hi
            cat_ref[s, :, 0:D] = hi
            cat_ref[s, :, D:2 * D] = mid
            cat_ref[s, :, 2 * D:] = lo
            cn_ref[s] = lax.dot_general(ones_row, cb * cb,
                                        (((1,), (1,)), ((), ())),
                                        preferred_element_type=jnp.float32,
                                        precision=lax.Precision.HIGHEST)

    iota_f = lax.broadcasted_iota(jnp.int32, (HN, K), 1).astype(jnp.float32)
    idx_cols = lax.broadcasted_iota(jnp.int32, (HN, 8), 1)

    def _chain(r0):
        r = r0
        rn = jnp.sum(r * r, axis=1, keepdims=True)
        qsum = jnp.zeros((HN, D), jnp.float32)
        idx_acc = jnp.zeros((HN, 8), jnp.int32)
        losses = []
        for s in range(NQ):
            r2b = (2.0 * r).astype(jnp.bfloat16)
            m2 = lax.dot_general(r2b, hi_ref[s], (((1,), (1,)), ((), ())),
                                 preferred_element_type=jnp.float32)
            d2 = (rn - m2) + cn_ref[s, 0:1, :]
            dmin = jnp.min(d2, axis=1, keepdims=True)
            idx_f = jnp.min(jnp.where(d2 == dmin, iota_f, jnp.float32(K)),
                            axis=1, keepdims=True)
            onehot = (iota_f == idx_f).astype(jnp.bfloat16)
            u = lax.dot_general(onehot, cat_ref[s], (((1,), (0,)), ((), ())),
                                preferred_element_type=jnp.float32)
            q = (u[:, 0:D] + u[:, D:2 * D]) + u[:, 2 * D:]
            r = r - q
            qsum = qsum + q
            rn = jnp.sum(r * r, axis=1, keepdims=True)
            losses.append(jnp.sum(rn))
            idx_acc = idx_acc + jnp.where(idx_cols == s,
                                          idx_f.astype(jnp.int32), 0)
        return qsum, idx_acc, losses

    r0s = [x_ref[h * HN:(h + 1) * HN, :] for h in range(BN // HN)]
    outs = [_chain(r0) for r0 in r0s]

    for h, (r0h, (qsum_h, idx_h, _)) in enumerate(zip(r0s, outs)):
        q_ref[h * HN:(h + 1) * HN, :] = r0h + (qsum_h - r0h)
        idx_ref[h * HN:(h + 1) * HN, :] = idx_h

    loss_rows = lax.broadcasted_iota(jnp.int32, (8, 128), 0)
    loss_cols = lax.broadcasted_iota(jnp.int32, (8, 128), 1)
    loss_contrib = jnp.zeros((8, 128), jnp.float32)
    for s in range(NQ):
        tot = outs[0][2][s]
        for o in outs[1:]:
            tot = tot + o[2][s]
        loss_contrib = loss_contrib + tot * jnp.where(
            (loss_rows == 0) & (loss_cols == s), 1.0, 0.0)
    loss_ref[...] = loss_contrib[None]


def kernel(x, codebooks):
    B, T, Dd = x.shape
    N = B * T
    G = N // BN
    flat = x.reshape(N, Dd)

    q_flat, idx_pack, loss_sums = pl.pallas_call(
        _rvq_block,
        grid=(G,),
        in_specs=[
            pl.BlockSpec((BN, D), lambda i: (i, 0)),
            pl.BlockSpec((NQ, K, D), lambda i: (0, 0, 0)),
        ],
        out_specs=[
            pl.BlockSpec((BN, D), lambda i: (i, 0)),
            pl.BlockSpec((BN, 8), lambda i: (i, 0)),
            pl.BlockSpec((1, 8, 128), lambda i: (i, 0, 0)),
        ],
        out_shape=[
            jax.ShapeDtypeStruct((N, D), jnp.float32),
            jax.ShapeDtypeStruct((N, 8), jnp.int32),
            jax.ShapeDtypeStruct((G, 8, 128), jnp.float32),
        ],
        scratch_shapes=[
            pltpu.VMEM((NQ, K, D), jnp.bfloat16),
            pltpu.VMEM((NQ, K, 3 * D), jnp.bfloat16),
            pltpu.VMEM((NQ, 8, K), jnp.float32),
        ],
        compiler_params=pltpu.CompilerParams(
            dimension_semantics=("arbitrary",),
        ),
    )(flat, codebooks)

    quantized = q_flat.reshape(B, T, Dd)
    indices = idx_pack[:, :NQ].reshape(B, T, NQ)
    vq_loss = 1.25 * jnp.sum(loss_sums[:, 0, :NQ]) / jnp.float32(N * Dd)
    losses = jnp.full((NQ,), vq_loss, dtype=jnp.float32)
    return quantized, indices, losses


# speedup vs baseline: 1.1440x; 1.0272x over previous
import jax
import jax.numpy as jnp
from jax import lax
from jax.experimental import pallas as pl
from jax.experimental.pallas import tpu as pltpu

NQ = 4
K = 1024
D = 256
BN = 2304
HN = BN // 2


def _rvq_block(x_ref, cb_ref, q_ref, idx_ref, loss_ref,
               hi_ref, cat_ref, cn_ref):
    @pl.when(pl.program_id(0) == 0)
    def _prep():
        ones_row = jnp.ones((8, D), jnp.float32)
        for s in range(NQ):
            cb = cb_ref[s]
            hi = cb.astype(jnp.bfloat16)
            rem = cb - hi.astype(jnp.float32)
            mid = rem.astype(jnp.bfloat16)
            lo = (rem - mid.astype(jnp.float32)).astype(jnp.bfloat16)
            hi_ref[s] = hi
            cat_ref[s, :, 0:D] = hi
            cat_ref[s, :, D:2 * D] = mid
            cat_ref[s, :, 2 * D:] = lo
            cn_ref[s] = lax.dot_general(ones_row, cb * cb,
                                        (((1,), (1,)), ((), ())),
                                        preferred_element_type=jnp.float32,
                                        precision=lax.Precision.HIGHEST)

    iota_f = lax.broadcasted_iota(jnp.int32, (HN, K), 1).astype(jnp.float32)
    idx_cols = lax.broadcasted_iota(jnp.int32, (HN, 8), 1)

    def _chain(r0):
        r = r0
        rn = jnp.sum(r * r, axis=1, keepdims=True)
        qsum = jnp.zeros((HN, D), jnp.float32)
        idx_acc = jnp.zeros((HN, 8), jnp.int32)
        losses = []
        for s in range(NQ):
            r2b = (2.0 * r).astype(jnp.bfloat16)
            m2 = lax.dot_general(r2b, hi_ref[s], (((1,), (1,)), ((), ())),
                                 preferred_element_type=jnp.float32)
            d2 = (rn - m2) + cn_ref[s, 0:1, :]
            dmin = jnp.min(d2, axis=1, keepdims=True)
            idx_f = jnp.min(jnp.where(d2 == dmin, iota_f, jnp.float32(K)),
                            axis=1, keepdims=True)
            onehot = (iota_f == idx_f).astype(jnp.bfloat16)
            u = lax.dot_general(onehot, cat_ref[s], (((1,), (0,)), ((), ())),
                                preferred_element_type=jnp.float32)
            q = (u[:, 0:D] + u[:, D:2 * D]) + u[:, 2 * D:]
            r = r - q
            qsum = qsum + q
            rn = jnp.sum(r * r, axis=1, keepdims=True)
            losses.append(jnp.sum(rn))
            idx_acc = idx_acc + jnp.where(idx_cols == s,
                                          idx_f.astype(jnp.int32), 0)
        return qsum, idx_acc, losses

    r0s = [x_ref[h * HN:(h + 1) * HN, :] for h in range(BN // HN)]
    outs = [_chain(r0) for r0 in r0s]

    for h, (r0h, (qsum_h, idx_h, _)) in enumerate(zip(r0s, outs)):
        q_ref[h * HN:(h + 1) * HN, :] = r0h + (qsum_h - r0h)
        idx_ref[h * HN:(h + 1) * HN, :] = idx_h

    loss_rows = lax.broadcasted_iota(jnp.int32, (8, 128), 0)
    loss_cols = lax.broadcasted_iota(jnp.int32, (8, 128), 1)
    loss_contrib = jnp.zeros((8, 128), jnp.float32)
    for s in range(NQ):
        tot = outs[0][2][s]
        for o in outs[1:]:
            tot = tot + o[2][s]
        loss_contrib = loss_contrib + tot * jnp.where(
            (loss_rows == 0) & (loss_cols == s), 1.0, 0.0)
    loss_ref[...] = loss_contrib[None]


def kernel(x, codebooks):
    B, T, Dd = x.shape
    N = B * T
    G = N // BN
    flat = x.reshape(N, Dd)

    q_flat, idx_pack, loss_sums = pl.pallas_call(
        _rvq_block,
        grid=(G,),
        in_specs=[
            pl.BlockSpec((BN, D), lambda i: (i, 0)),
            pl.BlockSpec((NQ, K, D), lambda i: (0, 0, 0)),
        ],
        out_specs=[
            pl.BlockSpec((BN, D), lambda i: (i, 0)),
            pl.BlockSpec((BN, 8), lambda i: (i, 0)),
            pl.BlockSpec((1, 8, 128), lambda i: (i, 0, 0)),
        ],
        out_shape=[
            jax.ShapeDtypeStruct((N, D), jnp.float32),
            jax.ShapeDtypeStruct((N, 8), jnp.int32),
            jax.ShapeDtypeStruct((G, 8, 128), jnp.float32),
        ],
        scratch_shapes=[
            pltpu.VMEM((NQ, K, D), jnp.bfloat16),
            pltpu.VMEM((NQ, K, 3 * D), jnp.bfloat16),
            pltpu.VMEM((NQ, 8, K), jnp.float32),
        ],
        compiler_params=pltpu.CompilerParams(
            dimension_semantics=("arbitrary",),
        ),
    )(flat, codebooks)

    quantized = q_flat.reshape(B, T, Dd)
    indices = idx_pack[:, :NQ].reshape(B, T, NQ)
    vq_loss = 1.25 * jnp.sum(loss_sums[:, 0, :NQ]) / jnp.float32(N * Dd)
    losses = jnp.full((NQ,), vq_loss, dtype=jnp.float32)
    return quantized, indices, losses
